# Initial kernel scaffold; baseline (speedup 1.0000x reference)
import jax
import jax.numpy as jnp
from jax.experimental import pallas as pl


def kernel(x, table, W, b):
    raise NotImplementedError("write your pallas kernel here")


# speedup vs baseline: 67.2790x; 67.2790x over previous
import functools

import jax
import jax.numpy as jnp
from jax import lax
from jax.experimental import pallas as pl
from jax.experimental.pallas import tpu as pltpu
from jax.experimental.pallas import tpu_sc as plsc

B = 16384
L = 50
D = 128
LANES = 16
NC = 2
NS = 16
NW = NC * NS
SPW = B // NW
GP = 2
ROWS_G = GP * L
GPW = SPW // GP
NBUF = 4
JG = LANES // GP
CC = GPW // JG


def _sc_body(x_hbm, w_hbm, b_hbm, table_hbm, out_hbm,
             idx_v, w_v, b_v, out_v,
             rows0, rows1, rows2, rows3,
             sem0, sem1, sem2, sem3):
  rows_bufs = (rows0, rows1, rows2, rows3)
  sems = (sem0, sem1, sem2, sem3)
  wid = lax.axis_index("s") * NC + lax.axis_index("c")

  pltpu.sync_copy(x_hbm.at[pl.ds(wid * GPW, GPW), :], idx_v)
  pltpu.sync_copy(w_hbm, w_v)
  pltpu.sync_copy(b_hbm, b_v)

  for j in range(NBUF):
    pltpu.async_copy(table_hbm.at[idx_v.at[j]], rows_bufs[j], sems[j])

  zeros = jnp.zeros((LANES,), jnp.float32)
  lane = lax.broadcasted_iota(jnp.int32, (LANES,), 0)
  bias = b_v[...]

  def lane_sum(a):
    for shift in (8, 4, 2, 1):
      perm = (lane + shift) & (LANES - 1)
      a = a + a.at[perm].get(mode="promise_in_bounds")
    return a

  def cc_body(cc, carry):
    ov = zeros
    for j in range(JG):
      buf = rows_bufs[j % NBUF]
      sem = sems[j % NBUF]
      g = cc * JG + j
      pltpu.make_async_copy(table_hbm.at[idx_v.at[g]], buf, sem).wait()

      def l_body(l, accs):
        a0, a1 = accs
        for k in range(D // LANES):
          ds = pl.ds(k * LANES, LANES)
          wv = w_v[l, ds]
          a0 = a0 + buf[l, ds] * wv
          a1 = a1 + buf[L + l, ds] * wv
        return (a0, a1)

      a0, a1 = lax.fori_loop(0, L, l_body, (zeros, zeros))
      ov = jnp.where(lane == GP * j, lane_sum(a0), ov)
      ov = jnp.where(lane == GP * j + 1, lane_sum(a1), ov)

      nxt = g + NBUF

      @pl.when(nxt < GPW)
      def _():
        pltpu.async_copy(table_hbm.at[idx_v.at[nxt]], buf, sem)

    out_v[pl.ds(cc * LANES, LANES)] = ov + bias
    return carry

  lax.fori_loop(0, CC, cc_body, 0)
  pltpu.sync_copy(out_v, out_hbm.at[pl.ds(wid * SPW, SPW)])


@jax.jit
def _run(x2, table, w2, b16):
  mesh = plsc.VectorSubcoreMesh(
      core_axis_name="c", subcore_axis_name="s",
      num_cores=NC, num_subcores=NS)
  kern = pl.kernel(
      _sc_body,
      out_type=jax.ShapeDtypeStruct((B,), jnp.float32),
      mesh=mesh,
      scratch_types=[
          pltpu.VMEM((GPW, ROWS_G), jnp.int32),
          pltpu.VMEM((L, D), jnp.float32),
          pltpu.VMEM((LANES,), jnp.float32),
          pltpu.VMEM((SPW,), jnp.float32),
          pltpu.VMEM((ROWS_G, D), jnp.float32),
          pltpu.VMEM((ROWS_G, D), jnp.float32),
          pltpu.VMEM((ROWS_G, D), jnp.float32),
          pltpu.VMEM((ROWS_G, D), jnp.float32),
          pltpu.SemaphoreType.DMA,
          pltpu.SemaphoreType.DMA,
          pltpu.SemaphoreType.DMA,
          pltpu.SemaphoreType.DMA,
      ],
  )
  return kern(x2, w2, b16, table)


def kernel(x, table, W, b):
  x2 = x.reshape(B * L // ROWS_G, ROWS_G)
  w2 = W.reshape(L, D)
  b16 = jnp.broadcast_to(b, (LANES,)).astype(jnp.float32)
  return _run(x2, table, w2, b16)
